# baseline (speedup 1.0000x reference)
import functools

import jax
import jax.numpy as jnp
from jax.experimental import pallas as pl
from jax.experimental.pallas import tpu as pltpu

_K = 8


def _candidates(x, k):
    r = x.shape[0]
    if k == 1:
        while r > _K:
            x = jnp.maximum(x[: r // 2], x[r // 2 :])
            r //= 2
        return [jnp.max(x, axis=0, keepdims=True)]
    if r <= _K:
        return [x]
    hi = jnp.maximum(x[: r // 2], x[r // 2 :])
    lo = jnp.minimum(x[: r // 2], x[r // 2 :])
    return _candidates(hi, k) + _candidates(lo, (k + 1) // 2)


def _topk_body(x_ref, o_ref, acc_ref, *, n_sb):
    sb = pl.program_id(1)

    @pl.when(sb == 0)
    def _():
        acc_ref[...] = jnp.full(acc_ref.shape, -jnp.inf, acc_ref.dtype)

    def _ordered_concat(pieces, pad_to_pow2=False):
        pieces = sorted(pieces, key=lambda p: -p.shape[0])
        n_pool = sum(p.shape[0] for p in pieces)
        if pad_to_pow2:
            n_pad = 1 << (n_pool - 1).bit_length()
            if n_pad > n_pool:
                pieces.append(
                    jnp.full((n_pad - n_pool, pieces[0].shape[1]), -jnp.inf,
                             pieces[0].dtype)
                )
        return jnp.concatenate(pieces, axis=0)

    cands = [acc_ref[...]] + _candidates(x_ref[0], _K)
    pool = _ordered_concat(cands, pad_to_pow2=True)
    x = _ordered_concat(_candidates(pool, _K))
    n = x.shape[0]
    rows = jax.lax.broadcasted_iota(jnp.int32, x.shape, 0)
    outs = []
    for _ in range(_K):
        m = jnp.max(x, axis=0)
        outs.append(m)
        idx = jnp.min(jnp.where(x == m[None, :], rows, n), axis=0)
        x = jnp.where(rows == idx[None, :], -jnp.inf, x)
    acc_ref[...] = jnp.stack(outs, axis=0)

    @pl.when(sb == n_sb - 1)
    def _():
        o_ref[0] = acc_ref[...].T


def _kmax(x, s_blk=4096, interpret=False):
    b, s, c = x.shape
    n_sb = s // s_blk
    out = pl.pallas_call(
        functools.partial(_topk_body, n_sb=n_sb),
        grid=(b, n_sb),
        in_specs=[pl.BlockSpec((1, s_blk, c), lambda i, j: (i, j, 0))],
        out_specs=pl.BlockSpec((1, c, _K), lambda i, j: (i, 0, 0)),
        out_shape=jax.ShapeDtypeStruct((b, c, _K), x.dtype),
        scratch_shapes=[pltpu.VMEM((_K, c), x.dtype)],
        compiler_params=pltpu.CompilerParams(
            dimension_semantics=("parallel", "arbitrary")
        ),
        interpret=interpret,
    )(x)
    return out.reshape(b, c * _K)


def kernel(inputs):
    return _kmax(inputs)


# speedup vs baseline: 1.0224x; 1.0224x over previous
import functools

import jax
import jax.numpy as jnp
from jax.experimental import pallas as pl
from jax.experimental.pallas import tpu as pltpu

_K = 8


def _candidates(x, k):
    r = x.shape[0]
    if k == 1:
        return [jnp.max(x, axis=0, keepdims=True)]
    if r <= _K:
        return [x]
    hi = jnp.maximum(x[: r // 2], x[r // 2 :])
    lo = jnp.minimum(x[: r // 2], x[r // 2 :])
    return _candidates(hi, k) + _candidates(lo, (k + 1) // 2)


def _topk_body(x_ref, o_ref, acc_ref, *, n_sb):
    sb = pl.program_id(1)

    @pl.when(sb == 0)
    def _():
        acc_ref[...] = jnp.full(acc_ref.shape, -jnp.inf, acc_ref.dtype)

    def _ordered_concat(pieces, pad_to_pow2=False):
        pieces = sorted(pieces, key=lambda p: -p.shape[0])
        n_pool = sum(p.shape[0] for p in pieces)
        if pad_to_pow2:
            n_pad = 1 << (n_pool - 1).bit_length()
            if n_pad > n_pool:
                pieces.append(
                    jnp.full((n_pad - n_pool, pieces[0].shape[1]), -jnp.inf,
                             pieces[0].dtype)
                )
        return jnp.concatenate(pieces, axis=0)

    cands = [acc_ref[...]] + _candidates(x_ref[0], _K)
    pool = _ordered_concat(cands, pad_to_pow2=True)
    x = _ordered_concat(_candidates(pool, _K))
    n = x.shape[0]
    rows = jax.lax.broadcasted_iota(jnp.int32, x.shape, 0)
    outs = []
    for _ in range(_K):
        m = jnp.max(x, axis=0)
        outs.append(m)
        idx = jnp.min(jnp.where(x == m[None, :], rows, n), axis=0)
        x = jnp.where(rows == idx[None, :], -jnp.inf, x)
    acc_ref[...] = jnp.stack(outs, axis=0)

    @pl.when(sb == n_sb - 1)
    def _():
        o_ref[0] = acc_ref[...].T


def _kmax(x, s_blk=4096, interpret=False):
    b, s, c = x.shape
    n_sb = s // s_blk
    out = pl.pallas_call(
        functools.partial(_topk_body, n_sb=n_sb),
        grid=(b, n_sb),
        in_specs=[pl.BlockSpec((1, s_blk, c), lambda i, j: (i, j, 0))],
        out_specs=pl.BlockSpec((1, c, _K), lambda i, j: (i, 0, 0)),
        out_shape=jax.ShapeDtypeStruct((b, c, _K), x.dtype),
        scratch_shapes=[pltpu.VMEM((_K, c), x.dtype)],
        compiler_params=pltpu.CompilerParams(
            dimension_semantics=("parallel", "arbitrary")
        ),
        interpret=interpret,
    )(x)
    return out.reshape(b, c * _K)


def kernel(inputs):
    return _kmax(inputs)


# speedup vs baseline: 1.0646x; 1.0412x over previous
import functools

import jax
import jax.numpy as jnp
from jax.experimental import pallas as pl
from jax.experimental.pallas import tpu as pltpu

_K = 8


def _candidates(x, k):
    r = x.shape[0]
    if k == 1:
        return [jnp.max(x, axis=0, keepdims=True)]
    if r <= _K:
        return [x]
    hi = jnp.maximum(x[: r // 2], x[r // 2 :])
    lo = jnp.minimum(x[: r // 2], x[r // 2 :])
    return _candidates(hi, k) + _candidates(lo, (k + 1) // 2)


def _topk_body(x_ref, o_ref, acc_ref, *, n_sb):
    sb = pl.program_id(1)

    @pl.when(sb == 0)
    def _():
        acc_ref[...] = jnp.full(acc_ref.shape, -jnp.inf, acc_ref.dtype)

    cands = [acc_ref[...]] + _candidates(x_ref[0], _K)
    pool = jnp.concatenate(cands, axis=0)
    n_pool = pool.shape[0]
    n_pad = 1 << (n_pool - 1).bit_length()
    if n_pad > n_pool:
        pad = jnp.full((n_pad - n_pool, pool.shape[1]), -jnp.inf, pool.dtype)
        pool = jnp.concatenate([pool, pad], axis=0)
    x = jnp.concatenate(_candidates(pool, _K), axis=0)
    n = x.shape[0]
    rows = jax.lax.broadcasted_iota(jnp.int32, x.shape, 0)
    outs = []
    for _ in range(_K):
        m = jnp.max(x, axis=0)
        outs.append(m)
        idx = jnp.min(jnp.where(x == m[None, :], rows, n), axis=0)
        x = jnp.where(rows == idx[None, :], -jnp.inf, x)
    acc_ref[...] = jnp.stack(outs, axis=0)

    @pl.when(sb == n_sb - 1)
    def _():
        o_ref[0] = acc_ref[...].T


def _kmax(x, s_blk=4096):
    b, s, c = x.shape
    n_sb = s // s_blk
    out = pl.pallas_call(
        functools.partial(_topk_body, n_sb=n_sb),
        grid=(b, n_sb),
        in_specs=[pl.BlockSpec((1, s_blk, c), lambda i, j: (i, j, 0))],
        out_specs=pl.BlockSpec((1, c, _K), lambda i, j: (i, 0, 0)),
        out_shape=jax.ShapeDtypeStruct((b, c, _K), x.dtype),
        scratch_shapes=[pltpu.VMEM((_K, c), x.dtype)],
        compiler_params=pltpu.CompilerParams(
            dimension_semantics=("parallel", "arbitrary")
        ),
    )(x)
    return out.reshape(b, c * _K)


def kernel(inputs):
    return _kmax(inputs)


# speedup vs baseline: 1.0763x; 1.0110x over previous
import functools

import jax
import jax.numpy as jnp
from jax.experimental import pallas as pl
from jax.experimental.pallas import tpu as pltpu

_K = 8


def _candidates(x, k):
    r = x.shape[0]
    if k == 1:
        return [jnp.max(x, axis=0, keepdims=True)]
    if r <= _K:
        return [x]
    hi = jnp.maximum(x[: r // 2], x[r // 2 :])
    lo = jnp.minimum(x[: r // 2], x[r // 2 :])
    return _candidates(hi, k) + _candidates(lo, (k + 1) // 2)


def _topk_body(x_ref, o_ref):
    pool = jnp.concatenate(_candidates(x_ref[0], _K), axis=0)
    n_pool = pool.shape[0]
    n_pad = 1 << (n_pool - 1).bit_length()
    if n_pad > n_pool:
        pad = jnp.full((n_pad - n_pool, pool.shape[1]), -jnp.inf, pool.dtype)
        pool = jnp.concatenate([pool, pad], axis=0)
    x = jnp.concatenate(_candidates(pool, _K), axis=0)
    n = x.shape[0]
    rows = jax.lax.broadcasted_iota(jnp.int32, x.shape, 0)
    outs = []
    for _ in range(_K):
        m = jnp.max(x, axis=0)
        outs.append(m)
        idx = jnp.min(jnp.where(x == m[None, :], rows, n), axis=0)
        x = jnp.where(rows == idx[None, :], -jnp.inf, x)
    o_ref[0] = jnp.stack(outs, axis=0).T


def _kmax(x, c_blk=384):
    b, s, c = x.shape
    n_cb = c // c_blk
    out = pl.pallas_call(
        _topk_body,
        grid=(b, n_cb),
        in_specs=[pl.BlockSpec((1, s, c_blk), lambda i, j: (i, 0, j))],
        out_specs=pl.BlockSpec((1, c_blk, _K), lambda i, j: (i, j, 0)),
        out_shape=jax.ShapeDtypeStruct((b, c, _K), x.dtype),
        compiler_params=pltpu.CompilerParams(
            dimension_semantics=("parallel", "arbitrary")
        ),
    )(x)
    return out.reshape(b, c * _K)


def kernel(inputs):
    return _kmax(inputs)
